# Initial kernel scaffold; baseline (speedup 1.0000x reference)
import jax
import jax.numpy as jnp
from jax.experimental import pallas as pl


def kernel(x, y, xs, ys, mean_y):
    raise NotImplementedError("write your pallas kernel here")


# speedup vs baseline: 4.7859x; 4.7859x over previous
import functools

import jax
import jax.numpy as jnp
from jax import lax
from jax.experimental import pallas as pl
from jax.experimental.pallas import tpu as pltpu
from jax.experimental.pallas import tpu_sc as plsc

_B = 1024
_V = 768
_D = 8192
_K = 4
_EPS = 1e-3

_DBLK = 512
_RBLK = 128
_FBLK = 256
_BIGI = 1 << 30


def _match_body(r_ref, xs_ref, idx_ref, maxval_ref, maxidx_ref, slotval_ref):
    j = pl.program_id(0)
    inner = lax.dot_general(
        r_ref[...].astype(jnp.bfloat16), xs_ref[...].astype(jnp.bfloat16),
        (((1,), (1,)), ((), ())),
        preferred_element_type=jnp.float32)
    base = j * _DBLK
    col = lax.broadcasted_iota(jnp.int32, (_B, _DBLK), 1)
    blkmax = jnp.max(inner, axis=1, keepdims=True)
    blkarg = jnp.min(jnp.where(inner == blkmax, col, _DBLK),
                     axis=1, keepdims=True) + base

    idx = idx_ref[...]
    parts = []
    for k in range(_K):
        rel = idx[:, k:k + 1] - base
        parts.append(jnp.sum(jnp.where(col == rel, inner, 0.0),
                             axis=1, keepdims=True))
    svblk = jnp.concatenate(parts, axis=1)

    @pl.when(j == 0)
    def _():
        maxval_ref[...] = blkmax
        maxidx_ref[...] = blkarg
        slotval_ref[...] = svblk

    @pl.when(j > 0)
    def _():
        cur = maxval_ref[...]
        upd = blkmax > cur
        maxval_ref[...] = jnp.where(upd, blkmax, cur)
        maxidx_ref[...] = jnp.where(upd, blkarg, maxidx_ref[...])
        slotval_ref[...] = slotval_ref[...] + svblk


def _match(r, xs, idx):
    return pl.pallas_call(
        _match_body,
        grid=(_D // _DBLK,),
        in_specs=[
            pl.BlockSpec((_B, _V), lambda j: (0, 0)),
            pl.BlockSpec((_DBLK, _V), lambda j: (j, 0)),
            pl.BlockSpec((_B, _K), lambda j: (0, 0)),
        ],
        out_specs=[
            pl.BlockSpec((_B, 1), lambda j: (0, 0)),
            pl.BlockSpec((_B, 1), lambda j: (0, 0)),
            pl.BlockSpec((_B, _K), lambda j: (0, 0)),
        ],
        out_shape=[
            jax.ShapeDtypeStruct((_B, 1), jnp.float32),
            jax.ShapeDtypeStruct((_B, 1), jnp.int32),
            jax.ShapeDtypeStruct((_B, _K), jnp.float32),
        ],
    )(r, xs, idx)


def _gather_rows(table, flat_idx):
    n = flat_idx.shape[0]
    info = plsc.get_sparse_core_info()
    nw = info.num_cores * info.num_subcores
    npw = n // nw
    mesh = plsc.VectorSubcoreMesh(core_axis_name="c", subcore_axis_name="s")

    @functools.partial(
        pl.kernel, mesh=mesh,
        out_type=jax.ShapeDtypeStruct((n, _V), jnp.float32),
        scratch_types=[
            pltpu.VMEM((npw,), jnp.int32),
            pltpu.VMEM((npw, _V), jnp.float32),
            pltpu.SemaphoreType.DMA,
        ],
    )
    def gk(table_hbm, idx_hbm, out_hbm, idx_v, rows_v, sem):
        wid = lax.axis_index("s") * info.num_cores + lax.axis_index("c")
        base = wid * npw
        pltpu.sync_copy(idx_hbm.at[pl.ds(base, npw)], idx_v)
        pltpu.async_copy(table_hbm.at[idx_v], rows_v, sem).wait()
        pltpu.sync_copy(rows_v, out_hbm.at[pl.ds(base, npw)])

    return gk(table, flat_idx)


def _bf(v):
    return v.astype(jnp.bfloat16).astype(jnp.float32)


def _update_body(ns, x_ref, g_ref, w_ref, idx_ref, *rest):
    atom_refs = rest[:ns]
    rnew_ref, wnew_ref = rest[ns], rest[ns + 1]
    x = x_ref[...]
    g = g_ref[...]
    w = w_ref[...]
    idx = idx_ref[...]
    atoms = [_bf(atom_refs[k][...]) for k in range(ns)]
    wprod = [_bf(w[:, k:k + 1]) * atoms[k] for k in range(ns)]
    gprod = [_bf(g[:, k:k + 1]) * atoms[k] for k in range(ns)]
    ranks = []
    for k in range(ns):
        rk = jnp.zeros_like(idx[:, :1])
        for j in range(ns):
            lt = (idx[:, j:j + 1] < idx[:, k:k + 1])
            tie = (idx[:, j:j + 1] == idx[:, k:k + 1]) & (j < k)
            rk = rk + jnp.where(lt | tie, 1, 0)
        ranks.append(rk)

    def _fold_ascending(prods):
        acc = jnp.zeros_like(x)
        for p in range(ns):
            pick = jnp.zeros_like(x)
            for k in range(ns):
                pick = pick + jnp.where(ranks[k] == p, prods[k], 0.0)
            acc = acc + pick
        return acc

    rec = _fold_ascending(wprod)
    c = _fold_ascending(gprod)
    r_old = x - rec
    csq = jnp.sum(c * c, axis=1, keepdims=True)
    cr = jnp.sum(c * r_old, axis=1, keepdims=True)
    step = cr / jnp.maximum(csq, _EPS)
    wnew = jnp.maximum(w + step * g, 0.0)
    w2prod = [_bf(wnew[:, k:k + 1]) * atoms[k] for k in range(ns)]
    rnew_ref[...] = x - _fold_ascending(w2prod)
    wnew_ref[...] = wnew


def _update(ns, x, g, w, idx, atoms):
    nb = _B // _RBLK
    atom_specs = [
        pl.BlockSpec((_RBLK, _V), functools.partial(lambda k, j: (k * nb + j, 0), k))
        for k in range(ns)
    ]
    return pl.pallas_call(
        functools.partial(_update_body, ns),
        grid=(nb,),
        in_specs=[
            pl.BlockSpec((_RBLK, _V), lambda j: (j, 0)),
            pl.BlockSpec((_RBLK, _K), lambda j: (j, 0)),
            pl.BlockSpec((_RBLK, _K), lambda j: (j, 0)),
            pl.BlockSpec((_RBLK, _K), lambda j: (j, 0)),
        ] + atom_specs,
        out_specs=[
            pl.BlockSpec((_RBLK, _V), lambda j: (j, 0)),
            pl.BlockSpec((_RBLK, _K), lambda j: (j, 0)),
        ],
        out_shape=[
            jax.ShapeDtypeStruct((_B, _V), jnp.float32),
            jax.ShapeDtypeStruct((_B, _K), jnp.float32),
        ],
    )(x, g, w, idx, *([atoms] * ns))


def _final_body(x_ref, y_ref, my_ref, r3_ref, w_ref, i_ref,
                a0, a1, a2, a3,
                wout_ref, iout_ref, xrec_ref, yrec_ref, l2_ref, sv_ref):
    j = pl.program_id(0)
    y = y_ref[...]
    w = w_ref[...]
    idx = i_ref[...]
    atoms = [_bf(a0[...]), _bf(a1[...]), _bf(a2[...]), _bf(a3[...])]
    yrec = jnp.zeros_like(y)
    for k in range(_K):
        yrec = yrec + _bf(w[:, k:k + 1]) * atoms[k]
    xrec_ref[...] = x_ref[...] - r3_ref[...]
    yrec_ref[...] = yrec
    dlt = y - yrec
    l2_ref[...] = jnp.sum(dlt * dlt, axis=1, keepdims=True)
    dv = y - my_ref[...]
    part = jnp.sum(dv * dv).reshape(1, 1)

    @pl.when(j == 0)
    def _():
        sv_ref[...] = part

    @pl.when(j > 0)
    def _():
        sv_ref[...] = sv_ref[...] + part

    pos = w > 0.0
    fidx = lax.broadcasted_iota(jnp.int32, (_FBLK, 2 * _K), 1)
    fvalid = jnp.ones((_FBLK, 2 * _K), dtype=jnp.bool_)
    for k in range(_K):
        fvalid = fvalid & ~(pos[:, k:k + 1] & (idx[:, k:k + 1] == fidx))
    cand_val = jnp.concatenate(
        [jnp.where(pos, w, -1.0),
         jnp.where(fvalid, 0.0, -1.0)], axis=1)
    cand_idx = jnp.concatenate(
        [jnp.where(pos, idx, _BIGI),
         jnp.where(fvalid, fidx, _BIGI)], axis=1)
    outw, outi = [], []
    for _ in range(_K):
        bv = jnp.max(cand_val, axis=1, keepdims=True)
        bi = jnp.min(jnp.where(cand_val == bv, cand_idx, _BIGI),
                     axis=1, keepdims=True)
        outw.append(bv)
        outi.append(bi)
        cand_val = jnp.where(cand_idx == bi, -2.0, cand_val)
    wout_ref[...] = jnp.concatenate(outw, axis=1)
    iout_ref[...] = jnp.concatenate(outi, axis=1)


def _final(x, y, mean_y, r3, w, idx, yatoms):
    nb = _B // _FBLK
    atom_specs = [
        pl.BlockSpec((_FBLK, _V), functools.partial(lambda k, j: (k * nb + j, 0), k))
        for k in range(_K)
    ]
    return pl.pallas_call(
        _final_body,
        grid=(nb,),
        in_specs=[
            pl.BlockSpec((_FBLK, _V), lambda j: (j, 0)),
            pl.BlockSpec((_FBLK, _V), lambda j: (j, 0)),
            pl.BlockSpec((1, _V), lambda j: (0, 0)),
            pl.BlockSpec((_FBLK, _V), lambda j: (j, 0)),
            pl.BlockSpec((_FBLK, _K), lambda j: (j, 0)),
            pl.BlockSpec((_FBLK, _K), lambda j: (j, 0)),
        ] + atom_specs,
        out_specs=[
            pl.BlockSpec((_FBLK, _K), lambda j: (j, 0)),
            pl.BlockSpec((_FBLK, _K), lambda j: (j, 0)),
            pl.BlockSpec((_FBLK, _V), lambda j: (j, 0)),
            pl.BlockSpec((_FBLK, _V), lambda j: (j, 0)),
            pl.BlockSpec((_FBLK, 1), lambda j: (j, 0)),
            pl.BlockSpec((1, 1), lambda j: (0, 0)),
        ],
        out_shape=[
            jax.ShapeDtypeStruct((_B, _K), jnp.float32),
            jax.ShapeDtypeStruct((_B, _K), jnp.int32),
            jax.ShapeDtypeStruct((_B, _V), jnp.float32),
            jax.ShapeDtypeStruct((_B, _V), jnp.float32),
            jax.ShapeDtypeStruct((_B, 1), jnp.float32),
            jax.ShapeDtypeStruct((1, 1), jnp.float32),
        ],
    )(x, y, mean_y, r3, w, idx, *([yatoms] * _K))


def kernel(x, y, xs, ys, mean_y):
    r = x
    idx = jnp.zeros((_B, _K), jnp.int32)
    w = jnp.zeros((_B, _K), jnp.float32)
    active = jnp.zeros((_B, _K), jnp.bool_)
    slot_id = jnp.arange(_K, dtype=jnp.int32)[None, :]

    for t in range(_K):
        maxval, maxidx, slotvals = _match(r, xs, idx)
        dup = active & (idx == maxidx)
        isdup = jnp.any(dup, axis=1, keepdims=True)
        ins = (~isdup) & (slot_id == t)
        idx = jnp.where(ins, maxidx, idx)
        active = active | ins
        sv = jnp.where(ins, maxval, slotvals)
        selected = active & ((w != 0.0) | (idx == maxidx))
        g = jnp.where(selected, sv, 0.0)
        ns = t + 1
        flat = idx[:, :ns].T.reshape(-1)
        atoms = _gather_rows(xs, flat)
        r, w = _update(ns, x, g, w, idx, atoms)

    yatoms = _gather_rows(ys, idx.T.reshape(-1))
    wout, iout, xrec, yrec, l2, svsum = _final(
        x, y, mean_y.reshape(1, _V), r, w, idx, yatoms)
    total_variance = svsum[0, 0] / _B
    losses = l2[:, 0] / total_variance
    return (wout, iout, xrec, yrec, losses)


# speedup vs baseline: 5.9895x; 1.2515x over previous
import functools

import jax
import jax.numpy as jnp
from jax import lax
from jax.experimental import pallas as pl
from jax.experimental.pallas import tpu as pltpu
from jax.experimental.pallas import tpu_sc as plsc

_B = 1024
_V = 768
_D = 8192
_K = 4
_EPS = 1e-3

_DBLK = 512
_RBLK = 128
_FBLK = 256
_BIGI = 1 << 30


def _bf(v):
    return v.astype(jnp.bfloat16).astype(jnp.float32)


def _match_body(r_ref, xs_ref, maxval_ref, maxidx_ref):
    j = pl.program_id(0)
    inner = lax.dot_general(
        r_ref[...], xs_ref[...], (((1,), (1,)), ((), ())),
        preferred_element_type=jnp.float32)
    base = j * _DBLK
    col = lax.broadcasted_iota(jnp.int32, (_B, _DBLK), 1)
    blkmax = jnp.max(inner, axis=1, keepdims=True)
    blkarg = jnp.min(jnp.where(inner == blkmax, col, _DBLK),
                     axis=1, keepdims=True) + base

    @pl.when(j == 0)
    def _():
        maxval_ref[...] = blkmax
        maxidx_ref[...] = blkarg

    @pl.when(j > 0)
    def _():
        cur = maxval_ref[...]
        upd = blkmax > cur
        maxval_ref[...] = jnp.where(upd, blkmax, cur)
        maxidx_ref[...] = jnp.where(upd, blkarg, maxidx_ref[...])


def _match(r_bf16, xs_bf16):
    return pl.pallas_call(
        _match_body,
        grid=(_D // _DBLK,),
        in_specs=[
            pl.BlockSpec((_B, _V), lambda j: (0, 0)),
            pl.BlockSpec((_DBLK, _V), lambda j: (j, 0)),
        ],
        out_specs=[
            pl.BlockSpec((_B, 1), lambda j: (0, 0)),
            pl.BlockSpec((_B, 1), lambda j: (0, 0)),
        ],
        out_shape=[
            jax.ShapeDtypeStruct((_B, 1), jnp.float32),
            jax.ShapeDtypeStruct((_B, 1), jnp.int32),
        ],
    )(r_bf16, xs_bf16)


def _gather_rows(table, flat_idx):
    n = flat_idx.shape[0]
    info = plsc.get_sparse_core_info()
    nw = info.num_cores * info.num_subcores
    npw = n // nw
    mesh = plsc.VectorSubcoreMesh(core_axis_name="c", subcore_axis_name="s")

    @functools.partial(
        pl.kernel, mesh=mesh,
        out_type=jax.ShapeDtypeStruct((n, _V), jnp.float32),
        scratch_types=[
            pltpu.VMEM((npw,), jnp.int32),
            pltpu.VMEM((npw, _V), jnp.float32),
            pltpu.SemaphoreType.DMA,
        ],
    )
    def gk(table_hbm, idx_hbm, out_hbm, idx_v, rows_v, sem):
        wid = lax.axis_index("s") * info.num_cores + lax.axis_index("c")
        base = wid * npw
        pltpu.sync_copy(idx_hbm.at[pl.ds(base, npw)], idx_v)
        pltpu.async_copy(table_hbm.at[idx_v], rows_v, sem).wait()
        pltpu.sync_copy(rows_v, out_hbm.at[pl.ds(base, npw)])

    return gk(table, flat_idx)


def _update_body(ns, last, x_ref, g_ref, w_ref, *rest):
    atom_refs = rest[:ns]
    out_refs = rest[ns:]
    x = x_ref[...]
    g = g_ref[...]
    w = w_ref[...]
    atoms = [_bf(atom_refs[k][...]) for k in range(ns)]
    c = jnp.zeros_like(x)
    rec = jnp.zeros_like(x)
    for k in range(ns):
        c = c + _bf(g[:, k:k + 1]) * atoms[k]
        rec = rec + _bf(w[:, k:k + 1]) * atoms[k]
    r_old = x - rec
    csq = jnp.sum(c * c, axis=1, keepdims=True)
    cr = jnp.sum(c * r_old, axis=1, keepdims=True)
    step = cr / jnp.maximum(csq, _EPS)
    wnew = jnp.maximum(w + step * g, 0.0)
    rec2 = jnp.zeros_like(x)
    for k in range(ns):
        rec2 = rec2 + _bf(wnew[:, k:k + 1]) * atoms[k]
    if last:
        xrec_ref, wnew_ref = out_refs
        xrec_ref[...] = rec2
        wnew_ref[...] = wnew
    else:
        rnew_ref, wnew_ref, svnext_ref = out_refs
        rnew_bf = (x - rec2).astype(jnp.bfloat16)
        rnew_ref[...] = rnew_bf
        wnew_ref[...] = wnew
        rnew_f = rnew_bf.astype(jnp.float32)
        svs = [jnp.sum(rnew_f * atoms[k], axis=1, keepdims=True)
               for k in range(ns)]
        svs += [jnp.zeros((_RBLK, 1), jnp.float32)] * (_K - ns)
        svnext_ref[...] = jnp.concatenate(svs, axis=1)


def _update(ns, last, x, g, w, atoms):
    nb = _B // _RBLK
    atom_specs = [
        pl.BlockSpec((_RBLK, _V), functools.partial(lambda k, j: (k * nb + j, 0), k))
        for k in range(ns)
    ]
    if last:
        out_specs = [
            pl.BlockSpec((_RBLK, _V), lambda j: (j, 0)),
            pl.BlockSpec((_RBLK, _K), lambda j: (j, 0)),
        ]
        out_shape = [
            jax.ShapeDtypeStruct((_B, _V), jnp.float32),
            jax.ShapeDtypeStruct((_B, _K), jnp.float32),
        ]
    else:
        out_specs = [
            pl.BlockSpec((_RBLK, _V), lambda j: (j, 0)),
            pl.BlockSpec((_RBLK, _K), lambda j: (j, 0)),
            pl.BlockSpec((_RBLK, _K), lambda j: (j, 0)),
        ]
        out_shape = [
            jax.ShapeDtypeStruct((_B, _V), jnp.bfloat16),
            jax.ShapeDtypeStruct((_B, _K), jnp.float32),
            jax.ShapeDtypeStruct((_B, _K), jnp.float32),
        ]
    return pl.pallas_call(
        functools.partial(_update_body, ns, last),
        grid=(nb,),
        in_specs=[
            pl.BlockSpec((_RBLK, _V), lambda j: (j, 0)),
            pl.BlockSpec((_RBLK, _K), lambda j: (j, 0)),
            pl.BlockSpec((_RBLK, _K), lambda j: (j, 0)),
        ] + atom_specs,
        out_specs=out_specs,
        out_shape=out_shape,
    )(x, g, w, *([atoms] * ns))


def _final_body(y_ref, my_ref, w_ref, i_ref,
                a0, a1, a2, a3,
                wout_ref, iout_ref, yrec_ref, l2_ref, sv_ref):
    j = pl.program_id(0)
    y = y_ref[...]
    w = w_ref[...]
    idx = i_ref[...]
    atoms = [_bf(a0[...]), _bf(a1[...]), _bf(a2[...]), _bf(a3[...])]
    yrec = jnp.zeros_like(y)
    for k in range(_K):
        yrec = yrec + _bf(w[:, k:k + 1]) * atoms[k]
    yrec_ref[...] = yrec
    dlt = y - yrec
    l2_ref[...] = jnp.sum(dlt * dlt, axis=1, keepdims=True)
    dv = y - my_ref[...]
    part = jnp.sum(dv * dv).reshape(1, 1)

    @pl.when(j == 0)
    def _():
        sv_ref[...] = part

    @pl.when(j > 0)
    def _():
        sv_ref[...] = sv_ref[...] + part

    pos = w > 0.0
    fidx = lax.broadcasted_iota(jnp.int32, (_FBLK, 2 * _K), 1)
    fvalid = jnp.ones((_FBLK, 2 * _K), dtype=jnp.bool_)
    for k in range(_K):
        fvalid = fvalid & ~(pos[:, k:k + 1] & (idx[:, k:k + 1] == fidx))
    cand_val = jnp.concatenate(
        [jnp.where(pos, w, -1.0),
         jnp.where(fvalid, 0.0, -1.0)], axis=1)
    cand_idx = jnp.concatenate(
        [jnp.where(pos, idx, _BIGI),
         jnp.where(fvalid, fidx, _BIGI)], axis=1)
    outw, outi = [], []
    for _ in range(_K):
        bv = jnp.max(cand_val, axis=1, keepdims=True)
        bi = jnp.min(jnp.where(cand_val == bv, cand_idx, _BIGI),
                     axis=1, keepdims=True)
        outw.append(bv)
        outi.append(bi)
        cand_val = jnp.where(cand_idx == bi, -2.0, cand_val)
    wout_ref[...] = jnp.concatenate(outw, axis=1)
    iout_ref[...] = jnp.concatenate(outi, axis=1)


def _final(y, mean_y, w, idx, yatoms):
    nb = _B // _FBLK
    atom_specs = [
        pl.BlockSpec((_FBLK, _V), functools.partial(lambda k, j: (k * nb + j, 0), k))
        for k in range(_K)
    ]
    return pl.pallas_call(
        _final_body,
        grid=(nb,),
        in_specs=[
            pl.BlockSpec((_FBLK, _V), lambda j: (j, 0)),
            pl.BlockSpec((1, _V), lambda j: (0, 0)),
            pl.BlockSpec((_FBLK, _K), lambda j: (j, 0)),
            pl.BlockSpec((_FBLK, _K), lambda j: (j, 0)),
        ] + atom_specs,
        out_specs=[
            pl.BlockSpec((_FBLK, _K), lambda j: (j, 0)),
            pl.BlockSpec((_FBLK, _K), lambda j: (j, 0)),
            pl.BlockSpec((_FBLK, _V), lambda j: (j, 0)),
            pl.BlockSpec((_FBLK, 1), lambda j: (j, 0)),
            pl.BlockSpec((1, 1), lambda j: (0, 0)),
        ],
        out_shape=[
            jax.ShapeDtypeStruct((_B, _K), jnp.float32),
            jax.ShapeDtypeStruct((_B, _K), jnp.int32),
            jax.ShapeDtypeStruct((_B, _V), jnp.float32),
            jax.ShapeDtypeStruct((_B, 1), jnp.float32),
            jax.ShapeDtypeStruct((1, 1), jnp.float32),
        ],
    )(y, mean_y, w, idx, *([yatoms] * _K))


def kernel(x, y, xs, ys, mean_y):
    xs_bf = xs.astype(jnp.bfloat16)
    r_bf = x.astype(jnp.bfloat16)
    idx = jnp.zeros((_B, _K), jnp.int32)
    w = jnp.zeros((_B, _K), jnp.float32)
    active = jnp.zeros((_B, _K), jnp.bool_)
    slotvals = jnp.zeros((_B, _K), jnp.float32)
    slot_id = jnp.arange(_K, dtype=jnp.int32)[None, :]

    for t in range(_K):
        maxval, maxidx = _match(r_bf, xs_bf)
        dup = active & (idx == maxidx)
        isdup = jnp.any(dup, axis=1, keepdims=True)
        ins = (~isdup) & (slot_id == t)
        idx = jnp.where(ins, maxidx, idx)
        active = active | ins
        sv = jnp.where(idx == maxidx, maxval, slotvals)
        selected = active & ((w != 0.0) | (idx == maxidx))
        g = jnp.where(selected, sv, 0.0)
        ns = t + 1
        flat = idx[:, :ns].T.reshape(-1)
        atoms = _gather_rows(xs, flat)
        last = (t == _K - 1)
        if last:
            xrec, w = _update(ns, True, x, g, w, atoms)
        else:
            r_bf, w, slotvals = _update(ns, False, x, g, w, atoms)

    yatoms = _gather_rows(ys, idx.T.reshape(-1))
    wout, iout, yrec, l2, svsum = _final(
        y, mean_y.reshape(1, _V), w, idx, yatoms)
    total_variance = svsum[0, 0] / _B
    losses = l2[:, 0] / total_variance
    return (wout, iout, xrec, yrec, losses)


# speedup vs baseline: 6.7215x; 1.1222x over previous
import functools

import jax
import jax.numpy as jnp
from jax import lax
from jax.experimental import pallas as pl
from jax.experimental.pallas import tpu as pltpu
from jax.experimental.pallas import tpu_sc as plsc

_B = 1024
_V = 768
_D = 8192
_K = 4
_EPS = 1e-3

_DBLK = 1024
_RBLK = 256
_FBLK = 256
_BIGI = 1 << 30


def _bf(v):
    return v.astype(jnp.bfloat16).astype(jnp.float32)


def _match_body(r_ref, xs_ref, maxval_ref, maxidx_ref):
    j = pl.program_id(0)
    inner = lax.dot_general(
        r_ref[...], xs_ref[...], (((1,), (1,)), ((), ())),
        preferred_element_type=jnp.float32)
    base = j * _DBLK
    col = lax.broadcasted_iota(jnp.int32, (_B, _DBLK), 1)
    blkmax = jnp.max(inner, axis=1, keepdims=True)
    blkarg = jnp.min(jnp.where(inner == blkmax, col, _DBLK),
                     axis=1, keepdims=True) + base

    @pl.when(j == 0)
    def _():
        maxval_ref[...] = blkmax
        maxidx_ref[...] = blkarg

    @pl.when(j > 0)
    def _():
        cur = maxval_ref[...]
        upd = blkmax > cur
        maxval_ref[...] = jnp.where(upd, blkmax, cur)
        maxidx_ref[...] = jnp.where(upd, blkarg, maxidx_ref[...])


def _match(r_bf16, xs_bf16):
    return pl.pallas_call(
        _match_body,
        grid=(_D // _DBLK,),
        in_specs=[
            pl.BlockSpec((_B, _V), lambda j: (0, 0)),
            pl.BlockSpec((_DBLK, _V), lambda j: (j, 0)),
        ],
        out_specs=[
            pl.BlockSpec((_B, 1), lambda j: (0, 0)),
            pl.BlockSpec((_B, 1), lambda j: (0, 0)),
        ],
        out_shape=[
            jax.ShapeDtypeStruct((_B, 1), jnp.float32),
            jax.ShapeDtypeStruct((_B, 1), jnp.int32),
        ],
    )(r_bf16, xs_bf16)


def _gather_rows(table, flat_idx):
    n = flat_idx.shape[0]
    info = plsc.get_sparse_core_info()
    nw = info.num_cores * info.num_subcores
    npw = n // nw
    mesh = plsc.VectorSubcoreMesh(core_axis_name="c", subcore_axis_name="s")

    @functools.partial(
        pl.kernel, mesh=mesh,
        out_type=jax.ShapeDtypeStruct((n, _V), jnp.float32),
        scratch_types=[
            pltpu.VMEM((npw,), jnp.int32),
            pltpu.VMEM((npw, _V), jnp.float32),
            pltpu.SemaphoreType.DMA,
        ],
    )
    def gk(table_hbm, idx_hbm, out_hbm, idx_v, rows_v, sem):
        wid = lax.axis_index("s") * info.num_cores + lax.axis_index("c")
        base = wid * npw
        pltpu.sync_copy(idx_hbm.at[pl.ds(base, npw)], idx_v)
        pltpu.async_copy(table_hbm.at[idx_v], rows_v, sem).wait()
        pltpu.sync_copy(rows_v, out_hbm.at[pl.ds(base, npw)])

    return gk(table, flat_idx)


def _update_body(ns, last, x_ref, g_ref, w_ref, *rest):
    atom_refs = rest[:ns]
    out_refs = rest[ns:]
    x = x_ref[...]
    g = g_ref[...]
    w = w_ref[...]
    atoms = [atom_refs[k][...].astype(jnp.float32) for k in range(ns)]
    c = jnp.zeros_like(x)
    rec = jnp.zeros_like(x)
    for k in range(ns):
        c = c + _bf(g[:, k:k + 1]) * atoms[k]
        rec = rec + _bf(w[:, k:k + 1]) * atoms[k]
    r_old = x - rec
    csq = jnp.sum(c * c, axis=1, keepdims=True)
    cr = jnp.sum(c * r_old, axis=1, keepdims=True)
    step = cr / jnp.maximum(csq, _EPS)
    wnew = jnp.maximum(w + step * g, 0.0)
    rec2 = jnp.zeros_like(x)
    for k in range(ns):
        rec2 = rec2 + _bf(wnew[:, k:k + 1]) * atoms[k]
    if last:
        xrec_ref, wnew_ref = out_refs
        xrec_ref[...] = rec2
        wnew_ref[...] = wnew
    else:
        rnew_ref, wnew_ref, svnext_ref = out_refs
        rnew_bf = (x - rec2).astype(jnp.bfloat16)
        rnew_ref[...] = rnew_bf
        wnew_ref[...] = wnew
        rnew_f = rnew_bf.astype(jnp.float32)
        svs = [jnp.sum(rnew_f * atoms[k], axis=1, keepdims=True)
               for k in range(ns)]
        svs += [jnp.zeros((_RBLK, 1), jnp.float32)] * (_K - ns)
        svnext_ref[...] = jnp.concatenate(svs, axis=1)


def _update(ns, last, x, g, w, atoms):
    nb = _B // _RBLK
    atom_specs = [pl.BlockSpec((_RBLK, _V), lambda j: (j, 0)) for _ in range(ns)]
    if last:
        out_specs = [
            pl.BlockSpec((_RBLK, _V), lambda j: (j, 0)),
            pl.BlockSpec((_RBLK, _K), lambda j: (j, 0)),
        ]
        out_shape = [
            jax.ShapeDtypeStruct((_B, _V), jnp.float32),
            jax.ShapeDtypeStruct((_B, _K), jnp.float32),
        ]
    else:
        out_specs = [
            pl.BlockSpec((_RBLK, _V), lambda j: (j, 0)),
            pl.BlockSpec((_RBLK, _K), lambda j: (j, 0)),
            pl.BlockSpec((_RBLK, _K), lambda j: (j, 0)),
        ]
        out_shape = [
            jax.ShapeDtypeStruct((_B, _V), jnp.bfloat16),
            jax.ShapeDtypeStruct((_B, _K), jnp.float32),
            jax.ShapeDtypeStruct((_B, _K), jnp.float32),
        ]
    return pl.pallas_call(
        functools.partial(_update_body, ns, last),
        grid=(nb,),
        in_specs=[
            pl.BlockSpec((_RBLK, _V), lambda j: (j, 0)),
            pl.BlockSpec((_RBLK, _K), lambda j: (j, 0)),
            pl.BlockSpec((_RBLK, _K), lambda j: (j, 0)),
        ] + atom_specs,
        out_specs=out_specs,
        out_shape=out_shape,
    )(x, g, w, *atoms)


def _final_body(y_ref, my_ref, w_ref, i_ref,
                a0, a1, a2, a3,
                wout_ref, iout_ref, yrec_ref, l2_ref, sv_ref):
    j = pl.program_id(0)
    y = y_ref[...]
    w = w_ref[...]
    idx = i_ref[...]
    atoms = [_bf(a0[...]), _bf(a1[...]), _bf(a2[...]), _bf(a3[...])]
    yrec = jnp.zeros_like(y)
    for k in range(_K):
        yrec = yrec + _bf(w[:, k:k + 1]) * atoms[k]
    yrec_ref[...] = yrec
    dlt = y - yrec
    l2_ref[...] = jnp.sum(dlt * dlt, axis=1, keepdims=True)
    dv = y - my_ref[...]
    part = jnp.sum(dv * dv).reshape(1, 1)

    @pl.when(j == 0)
    def _():
        sv_ref[...] = part

    @pl.when(j > 0)
    def _():
        sv_ref[...] = sv_ref[...] + part

    pos = w > 0.0
    fidx = lax.broadcasted_iota(jnp.int32, (_FBLK, 2 * _K), 1)
    fvalid = jnp.ones((_FBLK, 2 * _K), dtype=jnp.bool_)
    for k in range(_K):
        fvalid = fvalid & ~(pos[:, k:k + 1] & (idx[:, k:k + 1] == fidx))
    cand_val = jnp.concatenate(
        [jnp.where(pos, w, -1.0),
         jnp.where(fvalid, 0.0, -1.0)], axis=1)
    cand_idx = jnp.concatenate(
        [jnp.where(pos, idx, _BIGI),
         jnp.where(fvalid, fidx, _BIGI)], axis=1)
    outw, outi = [], []
    for _ in range(_K):
        bv = jnp.max(cand_val, axis=1, keepdims=True)
        bi = jnp.min(jnp.where(cand_val == bv, cand_idx, _BIGI),
                     axis=1, keepdims=True)
        outw.append(bv)
        outi.append(bi)
        cand_val = jnp.where(cand_idx == bi, -2.0, cand_val)
    wout_ref[...] = jnp.concatenate(outw, axis=1)
    iout_ref[...] = jnp.concatenate(outi, axis=1)


def _final(y, mean_y, w, idx, yatoms):
    nb = _B // _FBLK
    atom_specs = [
        pl.BlockSpec((_FBLK, _V), functools.partial(lambda k, j: (k * nb + j, 0), k))
        for k in range(_K)
    ]
    return pl.pallas_call(
        _final_body,
        grid=(nb,),
        in_specs=[
            pl.BlockSpec((_FBLK, _V), lambda j: (j, 0)),
            pl.BlockSpec((1, _V), lambda j: (0, 0)),
            pl.BlockSpec((_FBLK, _K), lambda j: (j, 0)),
            pl.BlockSpec((_FBLK, _K), lambda j: (j, 0)),
        ] + atom_specs,
        out_specs=[
            pl.BlockSpec((_FBLK, _K), lambda j: (j, 0)),
            pl.BlockSpec((_FBLK, _K), lambda j: (j, 0)),
            pl.BlockSpec((_FBLK, _V), lambda j: (j, 0)),
            pl.BlockSpec((_FBLK, 1), lambda j: (j, 0)),
            pl.BlockSpec((1, 1), lambda j: (0, 0)),
        ],
        out_shape=[
            jax.ShapeDtypeStruct((_B, _K), jnp.float32),
            jax.ShapeDtypeStruct((_B, _K), jnp.int32),
            jax.ShapeDtypeStruct((_B, _V), jnp.float32),
            jax.ShapeDtypeStruct((_B, 1), jnp.float32),
            jax.ShapeDtypeStruct((1, 1), jnp.float32),
        ],
    )(y, mean_y, w, idx, *([yatoms] * _K))


def kernel(x, y, xs, ys, mean_y):
    xs_bf = xs.astype(jnp.bfloat16)
    r_bf = x.astype(jnp.bfloat16)
    idx = jnp.zeros((_B, _K), jnp.int32)
    w = jnp.zeros((_B, _K), jnp.float32)
    active = jnp.zeros((_B, _K), jnp.bool_)
    slotvals = jnp.zeros((_B, _K), jnp.float32)
    slot_id = jnp.arange(_K, dtype=jnp.int32)[None, :]
    atoms_bf = []

    for t in range(_K):
        maxval, maxidx = _match(r_bf, xs_bf)
        dup = active & (idx == maxidx)
        isdup = jnp.any(dup, axis=1, keepdims=True)
        ins = (~isdup) & (slot_id == t)
        idx = jnp.where(ins, maxidx, idx)
        active = active | ins
        sv = jnp.where(idx == maxidx, maxval, slotvals)
        selected = active & ((w != 0.0) | (idx == maxidx))
        g = jnp.where(selected, sv, 0.0)
        ns = t + 1
        newatom = _gather_rows(xs, idx[:, t])
        atoms_bf.append(newatom.astype(jnp.bfloat16))
        last = (t == _K - 1)
        if last:
            xrec, w = _update(ns, True, x, g, w, atoms_bf)
        else:
            r_bf, w, slotvals = _update(ns, False, x, g, w, atoms_bf)

    yatoms = _gather_rows(ys, idx.T.reshape(-1))
    wout, iout, yrec, l2, svsum = _final(
        y, mean_y.reshape(1, _V), w, idx, yatoms)
    total_variance = svsum[0, 0] / _B
    losses = l2[:, 0] / total_variance
    return (wout, iout, xrec, yrec, losses)


# speedup vs baseline: 6.9255x; 1.0304x over previous
import functools

import jax
import jax.numpy as jnp
from jax import lax
from jax.experimental import pallas as pl
from jax.experimental.pallas import tpu as pltpu
from jax.experimental.pallas import tpu_sc as plsc

_B = 1024
_V = 768
_D = 8192
_K = 4
_EPS = 1e-3

_DBLK = 2048
_RBLK = 256
_FBLK = 256
_BIGI = 1 << 30


def _bf(v):
    return v.astype(jnp.bfloat16).astype(jnp.float32)


def _match_body(r_ref, xs_ref, maxval_ref, maxidx_ref):
    j = pl.program_id(0)
    inner = lax.dot_general(
        r_ref[...], xs_ref[...], (((1,), (1,)), ((), ())),
        preferred_element_type=jnp.float32)
    base = j * _DBLK
    col = lax.broadcasted_iota(jnp.int32, (_B, _DBLK), 1)
    blkmax = jnp.max(inner, axis=1, keepdims=True)
    blkarg = jnp.min(jnp.where(inner == blkmax, col, _DBLK),
                     axis=1, keepdims=True) + base

    @pl.when(j == 0)
    def _():
        maxval_ref[...] = blkmax
        maxidx_ref[...] = blkarg

    @pl.when(j > 0)
    def _():
        cur = maxval_ref[...]
        upd = blkmax > cur
        maxval_ref[...] = jnp.where(upd, blkmax, cur)
        maxidx_ref[...] = jnp.where(upd, blkarg, maxidx_ref[...])


def _match(r_bf16, xs_bf16):
    return pl.pallas_call(
        _match_body,
        grid=(_D // _DBLK,),
        in_specs=[
            pl.BlockSpec((_B, _V), lambda j: (0, 0)),
            pl.BlockSpec((_DBLK, _V), lambda j: (j, 0)),
        ],
        out_specs=[
            pl.BlockSpec((_B, 1), lambda j: (0, 0)),
            pl.BlockSpec((_B, 1), lambda j: (0, 0)),
        ],
        out_shape=[
            jax.ShapeDtypeStruct((_B, 1), jnp.float32),
            jax.ShapeDtypeStruct((_B, 1), jnp.int32),
        ],
    )(r_bf16, xs_bf16)


def _gather_rows(table, flat_idx):
    n = flat_idx.shape[0]
    info = plsc.get_sparse_core_info()
    nw = info.num_cores * info.num_subcores
    npw = n // nw
    mesh = plsc.VectorSubcoreMesh(core_axis_name="c", subcore_axis_name="s")

    @functools.partial(
        pl.kernel, mesh=mesh,
        out_type=jax.ShapeDtypeStruct((n, _V), jnp.float32),
        scratch_types=[
            pltpu.VMEM((npw,), jnp.int32),
            pltpu.VMEM((npw, _V), jnp.float32),
            pltpu.SemaphoreType.DMA,
        ],
    )
    def gk(table_hbm, idx_hbm, out_hbm, idx_v, rows_v, sem):
        wid = lax.axis_index("s") * info.num_cores + lax.axis_index("c")
        base = wid * npw
        pltpu.sync_copy(idx_hbm.at[pl.ds(base, npw)], idx_v)
        pltpu.async_copy(table_hbm.at[idx_v], rows_v, sem).wait()
        pltpu.sync_copy(rows_v, out_hbm.at[pl.ds(base, npw)])

    return gk(table, flat_idx)


def _update_body(ns, last, x_ref, g_ref, w_ref, *rest):
    atom_refs = rest[:ns]
    out_refs = rest[ns:]
    x = x_ref[...]
    g = g_ref[...]
    w = w_ref[...]
    atoms = [atom_refs[k][...].astype(jnp.float32) for k in range(ns)]
    c = jnp.zeros_like(x)
    rec = jnp.zeros_like(x)
    for k in range(ns):
        c = c + _bf(g[:, k:k + 1]) * atoms[k]
        rec = rec + _bf(w[:, k:k + 1]) * atoms[k]
    r_old = x - rec
    csq = jnp.sum(c * c, axis=1, keepdims=True)
    cr = jnp.sum(c * r_old, axis=1, keepdims=True)
    step = cr / jnp.maximum(csq, _EPS)
    wnew = jnp.maximum(w + step * g, 0.0)
    rec2 = jnp.zeros_like(x)
    for k in range(ns):
        rec2 = rec2 + _bf(wnew[:, k:k + 1]) * atoms[k]
    if last:
        xrec_ref, wnew_ref = out_refs
        xrec_ref[...] = rec2
        wnew_ref[...] = wnew
    else:
        rnew_ref, wnew_ref, svnext_ref = out_refs
        rnew_bf = (x - rec2).astype(jnp.bfloat16)
        rnew_ref[...] = rnew_bf
        wnew_ref[...] = wnew
        rnew_f = rnew_bf.astype(jnp.float32)
        svs = [jnp.sum(rnew_f * atoms[k], axis=1, keepdims=True)
               for k in range(ns)]
        svs += [jnp.zeros((_RBLK, 1), jnp.float32)] * (_K - ns)
        svnext_ref[...] = jnp.concatenate(svs, axis=1)


def _update(ns, last, x, g, w, atoms):
    nb = _B // _RBLK
    atom_specs = [pl.BlockSpec((_RBLK, _V), lambda j: (j, 0)) for _ in range(ns)]
    if last:
        out_specs = [
            pl.BlockSpec((_RBLK, _V), lambda j: (j, 0)),
            pl.BlockSpec((_RBLK, _K), lambda j: (j, 0)),
        ]
        out_shape = [
            jax.ShapeDtypeStruct((_B, _V), jnp.float32),
            jax.ShapeDtypeStruct((_B, _K), jnp.float32),
        ]
    else:
        out_specs = [
            pl.BlockSpec((_RBLK, _V), lambda j: (j, 0)),
            pl.BlockSpec((_RBLK, _K), lambda j: (j, 0)),
            pl.BlockSpec((_RBLK, _K), lambda j: (j, 0)),
        ]
        out_shape = [
            jax.ShapeDtypeStruct((_B, _V), jnp.bfloat16),
            jax.ShapeDtypeStruct((_B, _K), jnp.float32),
            jax.ShapeDtypeStruct((_B, _K), jnp.float32),
        ]
    return pl.pallas_call(
        functools.partial(_update_body, ns, last),
        grid=(nb,),
        in_specs=[
            pl.BlockSpec((_RBLK, _V), lambda j: (j, 0)),
            pl.BlockSpec((_RBLK, _K), lambda j: (j, 0)),
            pl.BlockSpec((_RBLK, _K), lambda j: (j, 0)),
        ] + atom_specs,
        out_specs=out_specs,
        out_shape=out_shape,
    )(x, g, w, *atoms)


def _final_body(y_ref, my_ref, w_ref, i_ref,
                a0, a1, a2, a3,
                wout_ref, iout_ref, yrec_ref, l2_ref, sv_ref):
    j = pl.program_id(0)
    y = y_ref[...]
    w = w_ref[...]
    idx = i_ref[...]
    atoms = [_bf(a0[...]), _bf(a1[...]), _bf(a2[...]), _bf(a3[...])]
    yrec = jnp.zeros_like(y)
    for k in range(_K):
        yrec = yrec + _bf(w[:, k:k + 1]) * atoms[k]
    yrec_ref[...] = yrec
    dlt = y - yrec
    l2_ref[...] = jnp.sum(dlt * dlt, axis=1, keepdims=True)
    dv = y - my_ref[...]
    part = jnp.sum(dv * dv).reshape(1, 1)

    @pl.when(j == 0)
    def _():
        sv_ref[...] = part

    @pl.when(j > 0)
    def _():
        sv_ref[...] = sv_ref[...] + part

    pos = w > 0.0
    fidx = lax.broadcasted_iota(jnp.int32, (_FBLK, 2 * _K), 1)
    fvalid = jnp.ones((_FBLK, 2 * _K), dtype=jnp.bool_)
    for k in range(_K):
        fvalid = fvalid & ~(pos[:, k:k + 1] & (idx[:, k:k + 1] == fidx))
    cand_val = jnp.concatenate(
        [jnp.where(pos, w, -1.0),
         jnp.where(fvalid, 0.0, -1.0)], axis=1)
    cand_idx = jnp.concatenate(
        [jnp.where(pos, idx, _BIGI),
         jnp.where(fvalid, fidx, _BIGI)], axis=1)
    outw, outi = [], []
    for _ in range(_K):
        bv = jnp.max(cand_val, axis=1, keepdims=True)
        bi = jnp.min(jnp.where(cand_val == bv, cand_idx, _BIGI),
                     axis=1, keepdims=True)
        outw.append(bv)
        outi.append(bi)
        cand_val = jnp.where(cand_idx == bi, -2.0, cand_val)
    wout_ref[...] = jnp.concatenate(outw, axis=1)
    iout_ref[...] = jnp.concatenate(outi, axis=1)


def _final(y, mean_y, w, idx, yatoms):
    nb = _B // _FBLK
    atom_specs = [
        pl.BlockSpec((_FBLK, _V), functools.partial(lambda k, j: (k * nb + j, 0), k))
        for k in range(_K)
    ]
    return pl.pallas_call(
        _final_body,
        grid=(nb,),
        in_specs=[
            pl.BlockSpec((_FBLK, _V), lambda j: (j, 0)),
            pl.BlockSpec((1, _V), lambda j: (0, 0)),
            pl.BlockSpec((_FBLK, _K), lambda j: (j, 0)),
            pl.BlockSpec((_FBLK, _K), lambda j: (j, 0)),
        ] + atom_specs,
        out_specs=[
            pl.BlockSpec((_FBLK, _K), lambda j: (j, 0)),
            pl.BlockSpec((_FBLK, _K), lambda j: (j, 0)),
            pl.BlockSpec((_FBLK, _V), lambda j: (j, 0)),
            pl.BlockSpec((_FBLK, 1), lambda j: (j, 0)),
            pl.BlockSpec((1, 1), lambda j: (0, 0)),
        ],
        out_shape=[
            jax.ShapeDtypeStruct((_B, _K), jnp.float32),
            jax.ShapeDtypeStruct((_B, _K), jnp.int32),
            jax.ShapeDtypeStruct((_B, _V), jnp.float32),
            jax.ShapeDtypeStruct((_B, 1), jnp.float32),
            jax.ShapeDtypeStruct((1, 1), jnp.float32),
        ],
    )(y, mean_y, w, idx, *([yatoms] * _K))


def kernel(x, y, xs, ys, mean_y):
    xs_bf = xs.astype(jnp.bfloat16)
    r_bf = x.astype(jnp.bfloat16)
    idx = jnp.zeros((_B, _K), jnp.int32)
    w = jnp.zeros((_B, _K), jnp.float32)
    active = jnp.zeros((_B, _K), jnp.bool_)
    slotvals = jnp.zeros((_B, _K), jnp.float32)
    slot_id = jnp.arange(_K, dtype=jnp.int32)[None, :]
    atoms_bf = []

    for t in range(_K):
        maxval, maxidx = _match(r_bf, xs_bf)
        dup = active & (idx == maxidx)
        isdup = jnp.any(dup, axis=1, keepdims=True)
        ins = (~isdup) & (slot_id == t)
        idx = jnp.where(ins, maxidx, idx)
        active = active | ins
        sv = jnp.where(idx == maxidx, maxval, slotvals)
        selected = active & ((w != 0.0) | (idx == maxidx))
        g = jnp.where(selected, sv, 0.0)
        ns = t + 1
        newatom = _gather_rows(xs, idx[:, t])
        atoms_bf.append(newatom.astype(jnp.bfloat16))
        last = (t == _K - 1)
        if last:
            xrec, w = _update(ns, True, x, g, w, atoms_bf)
        else:
            r_bf, w, slotvals = _update(ns, False, x, g, w, atoms_bf)

    yatoms = _gather_rows(ys, idx.T.reshape(-1))
    wout, iout, yrec, l2, svsum = _final(
        y, mean_y.reshape(1, _V), w, idx, yatoms)
    total_variance = svsum[0, 0] / _B
    losses = l2[:, 0] / total_variance
    return (wout, iout, xrec, yrec, losses)


# speedup vs baseline: 6.9460x; 1.0030x over previous
import functools

import jax
import jax.numpy as jnp
from jax import lax
from jax.experimental import pallas as pl
from jax.experimental.pallas import tpu as pltpu
from jax.experimental.pallas import tpu_sc as plsc

_B = 1024
_V = 768
_D = 8192
_K = 4
_EPS = 1e-3

_DBLK = 4096
_RBLK = 256
_FBLK = 256
_BIGI = 1 << 30


def _bf(v):
    return v.astype(jnp.bfloat16).astype(jnp.float32)


def _match_body(r_ref, xs_ref, maxval_ref, maxidx_ref):
    j = pl.program_id(0)
    inner = lax.dot_general(
        r_ref[...], xs_ref[...], (((1,), (1,)), ((), ())),
        preferred_element_type=jnp.float32)
    base = j * _DBLK
    col = lax.broadcasted_iota(jnp.int32, (_B, _DBLK), 1)
    blkmax = jnp.max(inner, axis=1, keepdims=True)
    blkarg = jnp.min(jnp.where(inner == blkmax, col, _DBLK),
                     axis=1, keepdims=True) + base

    @pl.when(j == 0)
    def _():
        maxval_ref[...] = blkmax
        maxidx_ref[...] = blkarg

    @pl.when(j > 0)
    def _():
        cur = maxval_ref[...]
        upd = blkmax > cur
        maxval_ref[...] = jnp.where(upd, blkmax, cur)
        maxidx_ref[...] = jnp.where(upd, blkarg, maxidx_ref[...])


def _match(r_bf16, xs_bf16):
    return pl.pallas_call(
        _match_body,
        grid=(_D // _DBLK,),
        in_specs=[
            pl.BlockSpec((_B, _V), lambda j: (0, 0)),
            pl.BlockSpec((_DBLK, _V), lambda j: (j, 0)),
        ],
        out_specs=[
            pl.BlockSpec((_B, 1), lambda j: (0, 0)),
            pl.BlockSpec((_B, 1), lambda j: (0, 0)),
        ],
        out_shape=[
            jax.ShapeDtypeStruct((_B, 1), jnp.float32),
            jax.ShapeDtypeStruct((_B, 1), jnp.int32),
        ],
    )(r_bf16, xs_bf16)


def _gather_rows(table, flat_idx):
    n = flat_idx.shape[0]
    info = plsc.get_sparse_core_info()
    nw = info.num_cores * info.num_subcores
    npw = n // nw
    mesh = plsc.VectorSubcoreMesh(core_axis_name="c", subcore_axis_name="s")

    @functools.partial(
        pl.kernel, mesh=mesh,
        out_type=jax.ShapeDtypeStruct((n, _V), jnp.float32),
        scratch_types=[
            pltpu.VMEM((npw,), jnp.int32),
            pltpu.VMEM((npw, _V), jnp.float32),
            pltpu.SemaphoreType.DMA,
        ],
    )
    def gk(table_hbm, idx_hbm, out_hbm, idx_v, rows_v, sem):
        wid = lax.axis_index("s") * info.num_cores + lax.axis_index("c")
        base = wid * npw
        pltpu.sync_copy(idx_hbm.at[pl.ds(base, npw)], idx_v)
        pltpu.async_copy(table_hbm.at[idx_v], rows_v, sem).wait()
        pltpu.sync_copy(rows_v, out_hbm.at[pl.ds(base, npw)])

    return gk(table, flat_idx)


def _update_body(ns, last, x_ref, g_ref, w_ref, *rest):
    atom_refs = rest[:ns]
    out_refs = rest[ns:]
    x = x_ref[...]
    g = g_ref[...]
    w = w_ref[...]
    atoms = [atom_refs[k][...].astype(jnp.float32) for k in range(ns)]
    c = jnp.zeros_like(x)
    rec = jnp.zeros_like(x)
    for k in range(ns):
        c = c + _bf(g[:, k:k + 1]) * atoms[k]
        rec = rec + _bf(w[:, k:k + 1]) * atoms[k]
    r_old = x - rec
    csq = jnp.sum(c * c, axis=1, keepdims=True)
    cr = jnp.sum(c * r_old, axis=1, keepdims=True)
    step = cr / jnp.maximum(csq, _EPS)
    wnew = jnp.maximum(w + step * g, 0.0)
    rec2 = jnp.zeros_like(x)
    for k in range(ns):
        rec2 = rec2 + _bf(wnew[:, k:k + 1]) * atoms[k]
    if last:
        xrec_ref, wnew_ref = out_refs
        xrec_ref[...] = rec2
        wnew_ref[...] = wnew
    else:
        rnew_ref, wnew_ref, svnext_ref = out_refs
        rnew_bf = (x - rec2).astype(jnp.bfloat16)
        rnew_ref[...] = rnew_bf
        wnew_ref[...] = wnew
        rnew_f = rnew_bf.astype(jnp.float32)
        svs = [jnp.sum(rnew_f * atoms[k], axis=1, keepdims=True)
               for k in range(ns)]
        svs += [jnp.zeros((_RBLK, 1), jnp.float32)] * (_K - ns)
        svnext_ref[...] = jnp.concatenate(svs, axis=1)


def _update(ns, last, x, g, w, atoms):
    nb = _B // _RBLK
    atom_specs = [pl.BlockSpec((_RBLK, _V), lambda j: (j, 0)) for _ in range(ns)]
    if last:
        out_specs = [
            pl.BlockSpec((_RBLK, _V), lambda j: (j, 0)),
            pl.BlockSpec((_RBLK, _K), lambda j: (j, 0)),
        ]
        out_shape = [
            jax.ShapeDtypeStruct((_B, _V), jnp.float32),
            jax.ShapeDtypeStruct((_B, _K), jnp.float32),
        ]
    else:
        out_specs = [
            pl.BlockSpec((_RBLK, _V), lambda j: (j, 0)),
            pl.BlockSpec((_RBLK, _K), lambda j: (j, 0)),
            pl.BlockSpec((_RBLK, _K), lambda j: (j, 0)),
        ]
        out_shape = [
            jax.ShapeDtypeStruct((_B, _V), jnp.bfloat16),
            jax.ShapeDtypeStruct((_B, _K), jnp.float32),
            jax.ShapeDtypeStruct((_B, _K), jnp.float32),
        ]
    return pl.pallas_call(
        functools.partial(_update_body, ns, last),
        grid=(nb,),
        in_specs=[
            pl.BlockSpec((_RBLK, _V), lambda j: (j, 0)),
            pl.BlockSpec((_RBLK, _K), lambda j: (j, 0)),
            pl.BlockSpec((_RBLK, _K), lambda j: (j, 0)),
        ] + atom_specs,
        out_specs=out_specs,
        out_shape=out_shape,
    )(x, g, w, *atoms)


def _final_body(y_ref, my_ref, w_ref, i_ref,
                a0, a1, a2, a3,
                wout_ref, iout_ref, yrec_ref, l2_ref, sv_ref):
    j = pl.program_id(0)
    y = y_ref[...]
    w = w_ref[...]
    idx = i_ref[...]
    atoms = [_bf(a0[...]), _bf(a1[...]), _bf(a2[...]), _bf(a3[...])]
    yrec = jnp.zeros_like(y)
    for k in range(_K):
        yrec = yrec + _bf(w[:, k:k + 1]) * atoms[k]
    yrec_ref[...] = yrec
    dlt = y - yrec
    l2_ref[...] = jnp.sum(dlt * dlt, axis=1, keepdims=True)
    dv = y - my_ref[...]
    part = jnp.sum(dv * dv).reshape(1, 1)

    @pl.when(j == 0)
    def _():
        sv_ref[...] = part

    @pl.when(j > 0)
    def _():
        sv_ref[...] = sv_ref[...] + part

    pos = w > 0.0
    fidx = lax.broadcasted_iota(jnp.int32, (_FBLK, 2 * _K), 1)
    fvalid = jnp.ones((_FBLK, 2 * _K), dtype=jnp.bool_)
    for k in range(_K):
        fvalid = fvalid & ~(pos[:, k:k + 1] & (idx[:, k:k + 1] == fidx))
    cand_val = jnp.concatenate(
        [jnp.where(pos, w, -1.0),
         jnp.where(fvalid, 0.0, -1.0)], axis=1)
    cand_idx = jnp.concatenate(
        [jnp.where(pos, idx, _BIGI),
         jnp.where(fvalid, fidx, _BIGI)], axis=1)
    outw, outi = [], []
    for _ in range(_K):
        bv = jnp.max(cand_val, axis=1, keepdims=True)
        bi = jnp.min(jnp.where(cand_val == bv, cand_idx, _BIGI),
                     axis=1, keepdims=True)
        outw.append(bv)
        outi.append(bi)
        cand_val = jnp.where(cand_idx == bi, -2.0, cand_val)
    wout_ref[...] = jnp.concatenate(outw, axis=1)
    iout_ref[...] = jnp.concatenate(outi, axis=1)


def _final(y, mean_y, w, idx, yatoms):
    nb = _B // _FBLK
    atom_specs = [
        pl.BlockSpec((_FBLK, _V), functools.partial(lambda k, j: (k * nb + j, 0), k))
        for k in range(_K)
    ]
    return pl.pallas_call(
        _final_body,
        grid=(nb,),
        in_specs=[
            pl.BlockSpec((_FBLK, _V), lambda j: (j, 0)),
            pl.BlockSpec((1, _V), lambda j: (0, 0)),
            pl.BlockSpec((_FBLK, _K), lambda j: (j, 0)),
            pl.BlockSpec((_FBLK, _K), lambda j: (j, 0)),
        ] + atom_specs,
        out_specs=[
            pl.BlockSpec((_FBLK, _K), lambda j: (j, 0)),
            pl.BlockSpec((_FBLK, _K), lambda j: (j, 0)),
            pl.BlockSpec((_FBLK, _V), lambda j: (j, 0)),
            pl.BlockSpec((_FBLK, 1), lambda j: (j, 0)),
            pl.BlockSpec((1, 1), lambda j: (0, 0)),
        ],
        out_shape=[
            jax.ShapeDtypeStruct((_B, _K), jnp.float32),
            jax.ShapeDtypeStruct((_B, _K), jnp.int32),
            jax.ShapeDtypeStruct((_B, _V), jnp.float32),
            jax.ShapeDtypeStruct((_B, 1), jnp.float32),
            jax.ShapeDtypeStruct((1, 1), jnp.float32),
        ],
    )(y, mean_y, w, idx, *([yatoms] * _K))


def kernel(x, y, xs, ys, mean_y):
    xs_bf = xs.astype(jnp.bfloat16)
    r_bf = x.astype(jnp.bfloat16)
    idx = jnp.zeros((_B, _K), jnp.int32)
    w = jnp.zeros((_B, _K), jnp.float32)
    active = jnp.zeros((_B, _K), jnp.bool_)
    slotvals = jnp.zeros((_B, _K), jnp.float32)
    slot_id = jnp.arange(_K, dtype=jnp.int32)[None, :]
    atoms_bf = []

    for t in range(_K):
        maxval, maxidx = _match(r_bf, xs_bf)
        dup = active & (idx == maxidx)
        isdup = jnp.any(dup, axis=1, keepdims=True)
        ins = (~isdup) & (slot_id == t)
        idx = jnp.where(ins, maxidx, idx)
        active = active | ins
        sv = jnp.where(idx == maxidx, maxval, slotvals)
        selected = active & ((w != 0.0) | (idx == maxidx))
        g = jnp.where(selected, sv, 0.0)
        ns = t + 1
        newatom = _gather_rows(xs, idx[:, t])
        atoms_bf.append(newatom.astype(jnp.bfloat16))
        last = (t == _K - 1)
        if last:
            xrec, w = _update(ns, True, x, g, w, atoms_bf)
        else:
            r_bf, w, slotvals = _update(ns, False, x, g, w, atoms_bf)

    yatoms = _gather_rows(ys, idx.T.reshape(-1))
    wout, iout, yrec, l2, svsum = _final(
        y, mean_y.reshape(1, _V), w, idx, yatoms)
    total_variance = svsum[0, 0] / _B
    losses = l2[:, 0] / total_variance
    return (wout, iout, xrec, yrec, losses)
